# baseline (speedup 1.0000x reference)
import functools

import jax
import jax.numpy as jnp
from jax.experimental import pallas as pl
from jax.experimental.pallas import tpu as pltpu

_BH = 512


def _h23(a):
    left = jnp.concatenate([a[:, :1], a[:, :-1]], axis=1)
    right = jnp.concatenate([a[:, 1:], a[:, -1:]], axis=1)
    h2 = jnp.maximum(left, right)
    return h2, jnp.maximum(h2, a)


def _nms_body(bh, x_ref, top_ref, bot_ref, o_ref):
    i = pl.program_id(1)
    ni = pl.num_programs(1)

    def row_h3(r):
        _, h3 = _h23(r)
        return h3

    cur = x_ref[0]
    h2, h3 = _h23(cur)
    above_row = jnp.where(i == 0, h3[0:1], row_h3(top_ref[0, 7:8, :]))
    below_row = jnp.where(
        i == ni - 1, h3[bh - 1 : bh], row_h3(bot_ref[0, 0:1, :])
    )
    above = jnp.concatenate([above_row, h3[: bh - 1]], axis=0)
    below = jnp.concatenate([h3[1:], below_row], axis=0)
    nmax = jnp.maximum(jnp.maximum(above, below), h2)
    o_ref[0] = jnp.where(cur > nmax, cur, 0.0)


def _nms(x, *, interpret=False):
    b, c, h, w = x.shape
    bc = b * c
    xr = x.reshape(bc, h, w)
    bh = min(_BH, h)
    ni = h // bh
    g8 = h // 8
    bh8 = bh // 8

    out = pl.pallas_call(
        functools.partial(_nms_body, bh),
        out_shape=jax.ShapeDtypeStruct((bc, h, w), x.dtype),
        grid=(bc, ni),
        in_specs=[
            pl.BlockSpec((1, bh, w), lambda b_, i: (b_, i, 0)),
            pl.BlockSpec(
                (1, 8, w), lambda b_, i: (b_, jnp.maximum(i * bh8 - 1, 0), 0)
            ),
            pl.BlockSpec(
                (1, 8, w),
                lambda b_, i: (b_, jnp.minimum((i + 1) * bh8, g8 - 1), 0),
            ),
        ],
        out_specs=pl.BlockSpec((1, bh, w), lambda b_, i: (b_, i, 0)),
        compiler_params=pltpu.CompilerParams(
            dimension_semantics=("parallel", "arbitrary"),
            vmem_limit_bytes=48 * 1024 * 1024,
        ),
        name="nms2d",
        interpret=interpret,
    )(xr, xr, xr)
    return out.reshape(b, c, h, w)


def kernel(x):
    return _nms(x)


# speedup vs baseline: 1.0790x; 1.0790x over previous
import functools

import jax
import jax.numpy as jnp
from jax.experimental import pallas as pl
from jax.experimental.pallas import tpu as pltpu

_BH = 1024


def _h23(a):
    left = jnp.concatenate([a[:, :1], a[:, :-1]], axis=1)
    right = jnp.concatenate([a[:, 1:], a[:, -1:]], axis=1)
    h2 = jnp.maximum(left, right)
    return h2, jnp.maximum(h2, a)


def _nms_body(bh, x_ref, top_ref, bot_ref, o_ref):
    i = pl.program_id(1)
    ni = pl.num_programs(1)

    def row_h3(r):
        _, h3 = _h23(r)
        return h3

    cur = x_ref[0]
    h2, h3 = _h23(cur)
    above_row = jnp.where(i == 0, h3[0:1], row_h3(top_ref[0, 7:8, :]))
    below_row = jnp.where(
        i == ni - 1, h3[bh - 1 : bh], row_h3(bot_ref[0, 0:1, :])
    )
    above = jnp.concatenate([above_row, h3[: bh - 1]], axis=0)
    below = jnp.concatenate([h3[1:], below_row], axis=0)
    nmax = jnp.maximum(jnp.maximum(above, below), h2)
    o_ref[0] = jnp.where(cur > nmax, cur, 0.0)


def _nms(x, *, interpret=False):
    b, c, h, w = x.shape
    bc = b * c
    xr = x.reshape(bc, h, w)
    bh = min(_BH, h)
    ni = h // bh
    g8 = h // 8
    bh8 = bh // 8

    out = pl.pallas_call(
        functools.partial(_nms_body, bh),
        out_shape=jax.ShapeDtypeStruct((bc, h, w), x.dtype),
        grid=(bc, ni),
        in_specs=[
            pl.BlockSpec((1, bh, w), lambda b_, i: (b_, i, 0)),
            pl.BlockSpec(
                (1, 8, w), lambda b_, i: (b_, jnp.maximum(i * bh8 - 1, 0), 0)
            ),
            pl.BlockSpec(
                (1, 8, w),
                lambda b_, i: (b_, jnp.minimum((i + 1) * bh8, g8 - 1), 0),
            ),
        ],
        out_specs=pl.BlockSpec((1, bh, w), lambda b_, i: (b_, i, 0)),
        compiler_params=pltpu.CompilerParams(
            dimension_semantics=("parallel", "arbitrary"),
            vmem_limit_bytes=52 * 1024 * 1024,
        ),
        name="nms2d",
        interpret=interpret,
    )(xr, xr, xr)
    return out.reshape(b, c, h, w)


def kernel(x):
    return _nms(x)


# speedup vs baseline: 1.0831x; 1.0037x over previous
import functools

import jax
import jax.numpy as jnp
from jax.experimental import pallas as pl
from jax.experimental.pallas import tpu as pltpu

_BH = 1024


def _h23(a):
    left = jnp.concatenate([a[:, :1], a[:, :-1]], axis=1)
    right = jnp.concatenate([a[:, 1:], a[:, -1:]], axis=1)
    h2 = jnp.maximum(left, right)
    return h2, jnp.maximum(h2, a)


def _nms_body(bh, x_ref, top_ref, bot_ref, o_ref):
    i = pl.program_id(1)
    ni = pl.num_programs(1)

    def row_h3(r):
        _, h3 = _h23(r)
        return h3

    cur = x_ref[0]
    top = jnp.where(i == 0, cur[0:1, :], top_ref[0, 7:8, :])
    bot = jnp.where(i == ni - 1, cur[bh - 1 : bh, :], bot_ref[0, 0:1, :])
    ext = jnp.concatenate([top, cur, bot], axis=0)
    h2, h3 = _h23(ext)
    nmax = jnp.maximum(
        jnp.maximum(h3[0:bh], h3[2 : bh + 2]), h2[1 : bh + 1]
    )
    o_ref[0] = jnp.where(cur > nmax, cur, 0.0)


def _nms(x, *, interpret=False):
    b, c, h, w = x.shape
    bc = b * c
    xr = x.reshape(bc, h, w)
    bh = min(_BH, h)
    ni = h // bh
    g8 = h // 8
    bh8 = bh // 8

    out = pl.pallas_call(
        functools.partial(_nms_body, bh),
        out_shape=jax.ShapeDtypeStruct((bc, h, w), x.dtype),
        grid=(bc, ni),
        in_specs=[
            pl.BlockSpec((1, bh, w), lambda b_, i: (b_, i, 0)),
            pl.BlockSpec(
                (1, 8, w), lambda b_, i: (b_, jnp.maximum(i * bh8 - 1, 0), 0)
            ),
            pl.BlockSpec(
                (1, 8, w),
                lambda b_, i: (b_, jnp.minimum((i + 1) * bh8, g8 - 1), 0),
            ),
        ],
        out_specs=pl.BlockSpec((1, bh, w), lambda b_, i: (b_, i, 0)),
        compiler_params=pltpu.CompilerParams(
            dimension_semantics=("parallel", "arbitrary"),
            vmem_limit_bytes=52 * 1024 * 1024,
        ),
        name="nms2d",
        interpret=interpret,
    )(xr, xr, xr)
    return out.reshape(b, c, h, w)


def kernel(x):
    return _nms(x)


# speedup vs baseline: 1.2117x; 1.1188x over previous
import functools

import jax
import jax.numpy as jnp
from jax.experimental import pallas as pl
from jax.experimental.pallas import tpu as pltpu

_BH = 1024
_CH = 8


def _h23(a):
    left = jnp.concatenate([a[:, :1], a[:, :-1]], axis=1)
    right = jnp.concatenate([a[:, 1:], a[:, -1:]], axis=1)
    h2 = jnp.maximum(left, right)
    return h2, jnp.maximum(h2, a)


def _nms_body(bh, ch, x_ref, top_ref, bot_ref, o_ref):
    i = pl.program_id(1)
    ni = pl.num_programs(1)

    def row_h3(r):
        _, h3 = _h23(r)
        return h3

    def emit(a, cur, h2, h3, above_row, below_row):
        above = jnp.concatenate([above_row, h3[:-1]], axis=0)
        below = jnp.concatenate([h3[1:], below_row], axis=0)
        nm = jnp.maximum(jnp.maximum(above, below), h2)
        o_ref[0, a : a + ch, :] = jnp.where(cur > nm, cur, 0.0)

    state = None
    for c in range(bh // ch):
        a = c * ch
        cur = x_ref[0, a : a + ch, :]
        h2, h3 = _h23(cur)
        if c == 0:
            above_row = jnp.where(
                i == 0, h3[0:1], row_h3(top_ref[0, 7:8, :])
            )
        else:
            pa, pcur, ph2, ph3, prow = state
            emit(pa, pcur, ph2, ph3, prow, h3[0:1])
            above_row = ph3[ch - 1 : ch]
        state = (a, cur, h2, h3, above_row)
    pa, pcur, ph2, ph3, prow = state
    below_row = jnp.where(
        i == ni - 1, ph3[ch - 1 : ch], row_h3(bot_ref[0, 0:1, :])
    )
    emit(pa, pcur, ph2, ph3, prow, below_row)


def _nms(x, *, interpret=False):
    b, c, h, w = x.shape
    bc = b * c
    xr = x.reshape(bc, h, w)
    bh = min(_BH, h)
    ni = h // bh
    ch = min(_CH, bh)
    g8 = h // 8
    bh8 = bh // 8

    out = pl.pallas_call(
        functools.partial(_nms_body, bh, ch),
        out_shape=jax.ShapeDtypeStruct((bc, h, w), x.dtype),
        grid=(bc, ni),
        in_specs=[
            pl.BlockSpec((1, bh, w), lambda b_, i: (b_, i, 0)),
            pl.BlockSpec(
                (1, 8, w), lambda b_, i: (b_, jnp.maximum(i * bh8 - 1, 0), 0)
            ),
            pl.BlockSpec(
                (1, 8, w),
                lambda b_, i: (b_, jnp.minimum((i + 1) * bh8, g8 - 1), 0),
            ),
        ],
        out_specs=pl.BlockSpec((1, bh, w), lambda b_, i: (b_, i, 0)),
        compiler_params=pltpu.CompilerParams(
            dimension_semantics=("parallel", "arbitrary"),
            vmem_limit_bytes=52 * 1024 * 1024,
        ),
        name="nms2d",
        interpret=interpret,
    )(xr, xr, xr)
    return out.reshape(b, c, h, w)


def kernel(x):
    return _nms(x)


# speedup vs baseline: 1.2212x; 1.0079x over previous
import functools

import jax
import jax.numpy as jnp
from jax.experimental import pallas as pl
from jax.experimental.pallas import tpu as pltpu

_BH = 1024
_CH = 8


def _h23(a):
    left = jnp.concatenate([a[:, :1], a[:, :-1]], axis=1)
    right = jnp.concatenate([a[:, 1:], a[:, -1:]], axis=1)
    h2 = jnp.maximum(left, right)
    return h2, jnp.maximum(h2, a)


def _nms_body(bh, ch, x_ref, top_ref, bot_ref, o_ref):
    i = pl.program_id(1)
    ni = pl.num_programs(1)

    def row_h3(r):
        _, h3 = _h23(r)
        return h3

    def emit(a, h2, h3, above_row, below_row):
        above = jnp.concatenate([above_row, h3[:-1]], axis=0)
        below = jnp.concatenate([h3[1:], below_row], axis=0)
        nm = jnp.maximum(jnp.maximum(above, below), h2)
        c2 = x_ref[0, a : a + ch, :]
        o_ref[0, a : a + ch, :] = jnp.where(c2 > nm, c2, 0.0)

    state = None
    for c in range(bh // ch):
        a = c * ch
        cur = x_ref[0, a : a + ch, :]
        h2, h3 = _h23(cur)
        if c == 0:
            above_row = jnp.where(
                i == 0, h3[0:1], row_h3(top_ref[0, 7:8, :])
            )
        else:
            pa, ph2, ph3, prow = state
            emit(pa, ph2, ph3, prow, h3[0:1])
            above_row = ph3[ch - 1 : ch]
        state = (a, h2, h3, above_row)
    pa, ph2, ph3, prow = state
    below_row = jnp.where(
        i == ni - 1, ph3[ch - 1 : ch], row_h3(bot_ref[0, 0:1, :])
    )
    emit(pa, ph2, ph3, prow, below_row)


def _nms(x, *, interpret=False):
    b, c, h, w = x.shape
    bc = b * c
    xr = x.reshape(bc, h, w)
    bh = min(_BH, h)
    ni = h // bh
    ch = min(_CH, bh)
    g8 = h // 8
    bh8 = bh // 8

    out = pl.pallas_call(
        functools.partial(_nms_body, bh, ch),
        out_shape=jax.ShapeDtypeStruct((bc, h, w), x.dtype),
        grid=(bc, ni),
        in_specs=[
            pl.BlockSpec((1, bh, w), lambda b_, i: (b_, i, 0)),
            pl.BlockSpec(
                (1, 8, w), lambda b_, i: (b_, jnp.maximum(i * bh8 - 1, 0), 0)
            ),
            pl.BlockSpec(
                (1, 8, w),
                lambda b_, i: (b_, jnp.minimum((i + 1) * bh8, g8 - 1), 0),
            ),
        ],
        out_specs=pl.BlockSpec((1, bh, w), lambda b_, i: (b_, i, 0)),
        compiler_params=pltpu.CompilerParams(
            dimension_semantics=("parallel", "arbitrary"),
            vmem_limit_bytes=52 * 1024 * 1024,
        ),
        name="nms2d",
        interpret=interpret,
    )(xr, xr, xr)
    return out.reshape(b, c, h, w)


def kernel(x):
    return _nms(x)


# speedup vs baseline: 1.2285x; 1.0060x over previous
import functools

import jax
import jax.numpy as jnp
from jax.experimental import pallas as pl
from jax.experimental.pallas import tpu as pltpu

_BH = 1024
_CH = 8


def _h23(a):
    left = jnp.concatenate([a[:, :1], a[:, :-1]], axis=1)
    right = jnp.concatenate([a[:, 1:], a[:, -1:]], axis=1)
    h2 = jnp.maximum(left, right)
    return h2, jnp.maximum(h2, a)


def _nms_body(bh, ch, x_ref, bot_ref, o_ref, tc_ref):
    i = pl.program_id(1)
    ni = pl.num_programs(1)

    def row_h3(r):
        _, h3 = _h23(r)
        return h3

    def emit(a, h2, h3, above_row, below_row):
        above = jnp.concatenate([above_row, h3[:-1]], axis=0)
        below = jnp.concatenate([h3[1:], below_row], axis=0)
        nm = jnp.maximum(jnp.maximum(above, below), h2)
        c2 = x_ref[0, a : a + ch, :]
        o_ref[0, a : a + ch, :] = jnp.where(c2 > nm, c2, 0.0)

    state = None
    for c in range(bh // ch):
        a = c * ch
        cur = x_ref[0, a : a + ch, :]
        h2, h3 = _h23(cur)
        if c == 0:
            above_row = jnp.where(i == 0, h3[0:1], tc_ref[0:1, :])
        else:
            pa, ph2, ph3, prow = state
            emit(pa, ph2, ph3, prow, h3[0:1])
            above_row = ph3[ch - 1 : ch]
        state = (a, h2, h3, above_row)
    pa, ph2, ph3, prow = state
    below_row = jnp.where(
        i == ni - 1, ph3[ch - 1 : ch], row_h3(bot_ref[0, 0:1, :])
    )
    emit(pa, ph2, ph3, prow, below_row)
    tc_ref[0:1, :] = ph3[ch - 1 : ch]


def _nms(x, *, interpret=False):
    b, c, h, w = x.shape
    bc = b * c
    xr = x.reshape(bc, h, w)
    bh = min(_BH, h)
    ni = h // bh
    ch = min(_CH, bh)
    g8 = h // 8
    bh8 = bh // 8

    out = pl.pallas_call(
        functools.partial(_nms_body, bh, ch),
        out_shape=jax.ShapeDtypeStruct((bc, h, w), x.dtype),
        grid=(bc, ni),
        in_specs=[
            pl.BlockSpec((1, bh, w), lambda b_, i: (b_, i, 0)),
            pl.BlockSpec(
                (1, 8, w),
                lambda b_, i: (b_, jnp.minimum((i + 1) * bh8, g8 - 1), 0),
            ),
        ],
        out_specs=pl.BlockSpec((1, bh, w), lambda b_, i: (b_, i, 0)),
        scratch_shapes=[pltpu.VMEM((8, w), jnp.float32)],
        compiler_params=pltpu.CompilerParams(
            dimension_semantics=("parallel", "arbitrary"),
            vmem_limit_bytes=52 * 1024 * 1024,
        ),
        name="nms2d",
        interpret=interpret,
    )(xr, xr)
    return out.reshape(b, c, h, w)


def kernel(x):
    return _nms(x)
